# Initial kernel scaffold; baseline (speedup 1.0000x reference)
import jax
import jax.numpy as jnp
from jax.experimental import pallas as pl


def kernel(x, edge_index, spatial_coords, ln_w, ln_b, W_self, b_self, W_nb, b_nb, W_red, b_red, beta):
    raise NotImplementedError("write your pallas kernel here")


# speedup vs baseline: 2.4348x; 2.4348x over previous
import functools

import jax
import jax.numpy as jnp
from jax import lax
from jax.experimental import pallas as pl
from jax.experimental.pallas import tpu as pltpu
from jax.experimental.pallas import tpu_sc as plsc

N = 10000
D = 128
RED = 32
NPAD = 10240
E = 160000
EPAD = 163840
NC = 2
NS = 16
NW = NC * NS
CH = 128
HALF = NPAD // NC
SH = HALF + 8
TH = N + 8
EPT_A = EPAD // NS
NCH_A = EPT_A // CH
EPT_B = EPAD // NW
NCH_B = EPT_B // CH
RADIUS = 1.0
NEG2R = -2.0 / (RADIUS + 1e-8)
F32 = jnp.float32
I32 = jnp.int32


def _sc_mesh():
    return plsc.VectorSubcoreMesh(
        core_axis_name="c", subcore_axis_name="s", num_cores=NC, num_subcores=NS
    )


def _sqrt16(a):
    i = plsc.bitcast(a, I32)
    i = 0x5F3759DF - (i >> 1)
    y = plsc.bitcast(i, F32)
    for _ in range(3):
        y = y * (1.5 - 0.5 * a * y * y)
    return a * y


def _zero_vmem_rows(ref, nrows, width):
    zero = jnp.zeros((16,), F32)

    def body(r, _):
        for j in range(width // 16):
            ref[r, pl.ds(j * 16, 16)] = zero
        return 0

    lax.fori_loop(0, nrows, body, 0)


@functools.cache
def _build_pass_a():
  return functools.partial(
    pl.kernel,
    out_type=(
        jax.ShapeDtypeStruct((NC, HALF, D), F32),
        jax.ShapeDtypeStruct((NC, HALF, 16), F32),
        jax.ShapeDtypeStruct((EPAD,), F32),
    ),
    mesh=_sc_mesh(),
    compiler_params=pltpu.CompilerParams(needs_layout_passes=False, use_tc_tiling_on_sc=False),
    scratch_types=[
        pltpu.VMEM((NPAD,), F32),
        pltpu.VMEM((NPAD,), F32),
        pltpu.VMEM((CH,), I32),
        pltpu.VMEM((CH,), I32),
        pltpu.VMEM((CH,), I32),
        pltpu.VMEM((CH,), F32),
        pltpu.VMEM((CH, D), F32),
        pltpu.VMEM((CH, D), F32),
        pltpu.VMEM((CH, 16), F32),
        pltpu.VMEM((CH, D), F32),
        pltpu.VMEM((320, 16), F32),
        pltpu.VMEM_SHARED((SH, D), F32),
        pltpu.VMEM_SHARED((SH, 16), F32),
        pltpu.SemaphoreType.DMA,
    ],
  )(_pass_a_body)


def _sc_pass_a(row, col, sx, sy, nb_all):
    return _build_pass_a()(row, col, sx, sy, nb_all)


def _pass_a_body(row_hbm, col_hbm, sx_hbm, sy_hbm, nb_hbm,
                 s_out, deg_out, dw_out,
                 sx_v, sy_v, rowc, colc, sidxc, dwc, nbrows, erows, ones_v,
                 zbuf, zdeg, s_sh, deg_sh, sem):
    c = lax.axis_index("c")
    s = lax.axis_index("s")

    _zero_vmem_rows(zbuf, CH, D)
    _zero_vmem_rows(zdeg, 320, 16)
    one = jnp.ones((16,), F32)

    def ones_body(r, _):
        ones_v[r, pl.ds(0, 16)] = one
        return 0

    lax.fori_loop(0, CH, ones_body, 0)
    pltpu.sync_copy(sx_hbm, sx_v)
    pltpu.sync_copy(sy_hbm, sy_v)
    r0 = s * (HALF // NS)
    pltpu.sync_copy(zbuf, s_sh.at[pl.ds(r0, CH)])
    pltpu.sync_copy(zbuf, s_sh.at[pl.ds(r0 + CH, CH)])
    pltpu.sync_copy(zbuf.at[pl.ds(0, 64)], s_sh.at[pl.ds(r0 + 2 * CH, 64)])
    pltpu.sync_copy(zdeg, deg_sh.at[pl.ds(r0, 320)])

    @pl.when(s == 0)
    def _():
        pltpu.sync_copy(zbuf.at[pl.ds(0, 8)], s_sh.at[pl.ds(HALF, 8)])
        pltpu.sync_copy(zdeg.at[pl.ds(0, 8)], deg_sh.at[pl.ds(HALF, 8)])

    plsc.subcore_barrier()

    def chunk(i, _):
        base = s * EPT_A + i * CH
        pltpu.sync_copy(row_hbm.at[pl.ds(base, CH)], rowc)
        pltpu.sync_copy(col_hbm.at[pl.ds(base, CH)], colc)
        cp = pltpu.async_copy(nb_hbm.at[colc], nbrows, sem)

        for g in range(CH // 16):
            ir = rowc[pl.ds(g * 16, 16)]
            ic = colc[pl.ds(g * 16, 16)]
            ax = plsc.load_gather(sx_v, [ir])
            ay = plsc.load_gather(sy_v, [ir])
            bx = plsc.load_gather(sx_v, [ic])
            by = plsc.load_gather(sy_v, [ic])
            dx = ax - bx
            dy = ay - by
            dist = _sqrt16(dx * dx + dy * dy)
            dwc[pl.ds(g * 16, 16)] = jnp.exp(dist * NEG2R)
            loc = ir - c * HALF
            inr = (loc >= 0) & (loc < HALF)
            sidxc[pl.ds(g * 16, 16)] = jnp.where(inr, loc, HALF)

        @pl.when(c == 0)
        def _():
            pltpu.sync_copy(dwc, dw_out.at[pl.ds(base, CH)])

        cp.wait()

        def egroup(g, _):
            dwv = dwc[pl.ds(g * 16, 16)]
            for k in range(16):
                sk = lax.squeeze(lax.slice(dwv, (k,), (k + 1,)), (0,))
                e = g * 16 + k
                for j in range(D // 16):
                    v = nbrows[e, pl.ds(j * 16, 16)] * sk
                    erows[e, pl.ds(j * 16, 16)] = jnp.exp(v)
            return 0

        lax.fori_loop(0, CH // 16, egroup, 0)

        pltpu.sync_copy(erows, s_sh.at[sidxc], add=True)
        pltpu.sync_copy(ones_v, deg_sh.at[sidxc], add=True)
        return 0

    lax.fori_loop(0, NCH_A, chunk, 0)
    plsc.subcore_barrier()

    pltpu.sync_copy(s_sh.at[pl.ds(r0, CH)], s_out.at[c, pl.ds(r0, CH)])
    pltpu.sync_copy(s_sh.at[pl.ds(r0 + CH, CH)], s_out.at[c, pl.ds(r0 + CH, CH)])
    pltpu.sync_copy(s_sh.at[pl.ds(r0 + 2 * CH, 64)],
                    s_out.at[c, pl.ds(r0 + 2 * CH, 64)])
    pltpu.sync_copy(deg_sh.at[pl.ds(r0, 320)], deg_out.at[c, pl.ds(r0, 320)])


@functools.cache
def _build_pass_b():
  return functools.partial(
    pl.kernel,
    out_type=jax.ShapeDtypeStruct((NC, HALF, D), F32),
    mesh=_sc_mesh(),
    compiler_params=pltpu.CompilerParams(needs_layout_passes=False, use_tc_tiling_on_sc=False),
    scratch_types=[
        pltpu.VMEM((CH,), I32),
        pltpu.VMEM((CH,), I32),
        pltpu.VMEM((CH,), I32),
        pltpu.VMEM((CH,), F32),
        pltpu.VMEM((CH, D), F32),
        pltpu.VMEM((CH, D), F32),
        pltpu.VMEM((CH, D), F32),
        pltpu.VMEM((CH, D), F32),
        pltpu.VMEM((CH, D), F32),
        pltpu.VMEM_SHARED((SH, D), F32),
        pltpu.SemaphoreType.DMA,
    ],
  )(_pass_b_body)


def _sc_pass_b(row, col, dw, nb_all, tgt, invd):
    return _build_pass_b()(row, col, dw, nb_all, tgt, invd)


def _pass_b_body(row_hbm, col_hbm, dw_hbm, nb_hbm, tgt_hbm, inv_hbm,
                 t_out,
                 rowc, colc, sidxc, dwc, nbrows, tgtrows, invrows, wrows,
                 zbuf, t_sh, sem):
    c = lax.axis_index("c")
    s = lax.axis_index("s")

    _zero_vmem_rows(zbuf, CH, D)
    r0 = s * (HALF // NS)
    pltpu.sync_copy(zbuf, t_sh.at[pl.ds(r0, CH)])
    pltpu.sync_copy(zbuf, t_sh.at[pl.ds(r0 + CH, CH)])
    pltpu.sync_copy(zbuf.at[pl.ds(0, 64)], t_sh.at[pl.ds(r0 + 2 * CH, 64)])

    @pl.when(s == 0)
    def _():
        pltpu.sync_copy(zbuf.at[pl.ds(0, 8)], t_sh.at[pl.ds(HALF, 8)])

    plsc.subcore_barrier()

    def chunk(i, _):
        base = s * EPT_A + i * CH
        pltpu.sync_copy(row_hbm.at[pl.ds(base, CH)], rowc)
        pltpu.sync_copy(col_hbm.at[pl.ds(base, CH)], colc)
        pltpu.sync_copy(dw_hbm.at[pl.ds(base, CH)], dwc)
        c1 = pltpu.async_copy(nb_hbm.at[colc], nbrows, sem)
        c2 = pltpu.async_copy(tgt_hbm.at[colc], tgtrows, sem)
        c3 = pltpu.async_copy(inv_hbm.at[rowc], invrows, sem)
        for g in range(CH // 16):
            v = rowc[pl.ds(g * 16, 16)]
            loc = v - c * HALF
            inr = (loc >= 0) & (loc < HALF)
            sidxc[pl.ds(g * 16, 16)] = jnp.where(inr, loc, HALF)
        c1.wait()
        c2.wait()
        c3.wait()

        thr = jnp.full((16,), 0.01, F32)
        zero = jnp.zeros((16,), F32)

        def egroup(g, _):
            dwv = dwc[pl.ds(g * 16, 16)]
            for k in range(16):
                sk = lax.squeeze(lax.slice(dwv, (k,), (k + 1,)), (0,))
                e = g * 16 + k
                for j in range(D // 16):
                    w = jnp.exp(nbrows[e, pl.ds(j * 16, 16)] * sk)
                    w = w * invrows[e, pl.ds(j * 16, 16)]
                    w = jnp.where(w >= thr, w, zero)
                    wrows[e, pl.ds(j * 16, 16)] = w * tgtrows[e, pl.ds(j * 16, 16)]
            return 0

        lax.fori_loop(0, CH // 16, egroup, 0)
        pltpu.sync_copy(wrows, t_sh.at[sidxc], add=True)
        return 0

    lax.fori_loop(0, NCH_A, chunk, 0)
    plsc.subcore_barrier()

    pltpu.sync_copy(t_sh.at[pl.ds(r0, CH)], t_out.at[c, pl.ds(r0, CH)])
    pltpu.sync_copy(t_sh.at[pl.ds(r0 + CH, CH)], t_out.at[c, pl.ds(r0 + CH, CH)])
    pltpu.sync_copy(t_sh.at[pl.ds(r0 + 2 * CH, 64)],
                    t_out.at[c, pl.ds(r0 + 2 * CH, 64)])


_BLK1 = 1024


def _tc_stage1_body(x_ref, lnw_ref, lnb_ref, ws_ref, bs_ref, wn_ref, bn_ref,
                    tgt_ref, nb_ref, eself_ref):
    xb = x_ref[...]
    mu = jnp.mean(xb, axis=-1, keepdims=True)
    var = jnp.mean((xb - mu) ** 2, axis=-1, keepdims=True)
    tgt = (xb - mu) / jnp.sqrt(var + 1e-5) * lnw_ref[...] + lnb_ref[...]
    tgt_ref[...] = tgt
    self_s = jnp.dot(tgt, ws_ref[...], preferred_element_type=F32) + bs_ref[...]
    eself_ref[...] = jnp.exp(self_s)
    nb_ref[...] = jnp.dot(tgt, wn_ref[...], preferred_element_type=F32) + bn_ref[...]


def _tc_stage1(x_pad, ln_w, ln_b, w_self, b_self, w_nb, b_nb):
    g = NPAD // _BLK1
    vec = pl.BlockSpec((1, D), lambda i: (0, 0))
    mat = pl.BlockSpec((D, D), lambda i: (0, 0))
    blk = pl.BlockSpec((_BLK1, D), lambda i: (i, 0))
    return pl.pallas_call(
        _tc_stage1_body,
        grid=(g,),
        in_specs=[blk, vec, vec, mat, vec, mat, vec],
        out_specs=(blk, blk, blk),
        out_shape=(
            jax.ShapeDtypeStruct((NPAD, D), F32),
            jax.ShapeDtypeStruct((NPAD, D), F32),
            jax.ShapeDtypeStruct((NPAD, D), F32),
        ),
    )(x_pad, ln_w.reshape(1, D), ln_b.reshape(1, D), w_self,
      b_self.reshape(1, D), w_nb, b_nb.reshape(1, D))


def _tc_stage3_body(eself_ref, s_ref, pcb_ref, pads_ref, inv_ref):
    exp_pad = jnp.exp(pads_ref[...])
    denom = eself_ref[...] + s_ref[...] + pcb_ref[...] * exp_pad
    inv_ref[...] = 1.0 / denom


def _tc_stage3(eself, s_full, pc_b, pad_score):
    g = NPAD // _BLK1
    blk = pl.BlockSpec((_BLK1, D), lambda i: (i, 0))
    return pl.pallas_call(
        _tc_stage3_body,
        grid=(g,),
        in_specs=[blk, blk, blk, pl.BlockSpec((1, D), lambda i: (0, 0))],
        out_specs=blk,
        out_shape=jax.ShapeDtypeStruct((NPAD, D), F32),
    )(eself, s_full, pc_b, pad_score)


def _tc_stage5_body(tgt_ref, eself_ref, inv_ref, t_ref, pcb_ref, pads_ref,
                    lnb_ref, wred_ref, bred_ref, beta_ref, out_ref):
    invd = inv_ref[...]
    self_w = eself_ref[...] * invd
    self_w = jnp.where(self_w >= 0.01, self_w, 0.0)
    pad_w = jnp.exp(pads_ref[...]) * invd
    pad_w = jnp.where(pad_w >= 0.01, pad_w, 0.0)
    nb_sum = t_ref[...] + pcb_ref[...] * (pad_w * lnb_ref[...])
    bb = beta_ref[...]
    ctx = bb * (self_w * tgt_ref[...]) + (1.0 - bb) * nb_sum
    red = jnp.dot(ctx, wred_ref[...], preferred_element_type=F32) + bred_ref[...]
    out_ref[...] = jnp.where(red >= 0.0, red, red * 0.0001)


def _tc_stage5(tgt, eself, invd, t_full, pc_b, pad_score, ln_b, w_red, b_red,
               beta):
    g = NPAD // _BLK1
    blk = pl.BlockSpec((_BLK1, D), lambda i: (i, 0))
    vec = pl.BlockSpec((1, D), lambda i: (0, 0))
    return pl.pallas_call(
        _tc_stage5_body,
        grid=(g,),
        in_specs=[blk, blk, blk, blk,
                  blk, vec, vec,
                  pl.BlockSpec((D, RED), lambda i: (0, 0)),
                  pl.BlockSpec((1, RED), lambda i: (0, 0)),
                  pl.BlockSpec((1, 1), lambda i: (0, 0))],
        out_specs=pl.BlockSpec((_BLK1, RED), lambda i: (i, 0)),
        out_shape=jax.ShapeDtypeStruct((NPAD, RED), F32),
    )(tgt, eself, invd, t_full, pc_b, pad_score, ln_b.reshape(1, D), w_red,
      b_red.reshape(1, RED), beta.reshape(1, 1))


def kernel(x, edge_index, spatial_coords, ln_w, ln_b, W_self, b_self, W_nb,
           b_nb, W_red, b_red, beta):
    x_pad = jnp.pad(x, ((0, NPAD - N), (0, 0)))
    pad_edge = jnp.full((EPAD - E,), NPAD - 1, I32)
    row = jnp.concatenate([edge_index[0], pad_edge])
    col = jnp.concatenate([edge_index[1], pad_edge])
    sx = jnp.pad(spatial_coords[:, 0], (0, NPAD - N))
    sy = jnp.pad(spatial_coords[:, 1], (0, NPAD - N))

    tgt, nb_all, eself = _tc_stage1(x_pad, ln_w, ln_b, W_self, b_self, W_nb,
                                    b_nb)
    pad_score = lax.slice(nb_all, (N, 0), (N + 1, D))

    s_halves, deg_halves, dw = _sc_pass_a(row, col, sx, sy, nb_all)
    s_full = jnp.concatenate([s_halves[0], s_halves[1]], axis=0)

    deg = jnp.concatenate([deg_halves[0, :, 0], deg_halves[1, :, 0]])[:N]
    pad_count = jnp.max(deg) - deg
    pc_b = jnp.broadcast_to(
        jnp.pad(pad_count, (0, NPAD - N))[:, None], (NPAD, D))

    invd = _tc_stage3(eself, s_full, pc_b, pad_score)

    t_halves = _sc_pass_b(row, col, dw, nb_all, tgt, invd)
    t_full = jnp.concatenate([t_halves[0], t_halves[1]], axis=0)

    out = _tc_stage5(tgt, eself, invd, t_full, pc_b, pad_score, ln_b, W_red,
                     b_red, beta)
    return out[:N]
